# baseline (speedup 1.0000x reference)
import jax
import jax.numpy as jnp
from jax.experimental import pallas as pl
from jax.experimental.pallas import tpu as pltpu

B, N, M, D = 4, 4096, 4096, 3
BN = 2048
NB = N // BN
STEPS = B * NB


def _chamfer_kernel(a_ref, bt_ref, out_ref, rowacc_ref, colmin_ref):
    s = pl.program_id(0)
    n_idx = jax.lax.rem(s, NB)
    b_par = jax.lax.rem(jax.lax.div(s, NB), 2)

    a = a_ref[0]
    bt = bt_ref[0]

    a2 = jnp.sum(a * a, axis=1, keepdims=True)
    b2 = jnp.sum(bt * bt, axis=0, keepdims=True)

    a2h = a2.astype(jnp.bfloat16)
    a2l = (a2 - a2h.astype(jnp.float32)).astype(jnp.bfloat16)
    b2h = b2.astype(jnp.bfloat16)
    b2l = (b2 - b2h.astype(jnp.float32)).astype(jnp.bfloat16)

    ones_a = jnp.ones((BN, 1), jnp.bfloat16)
    ones_b = jnp.ones((2, M), jnp.bfloat16)

    a_ext = jnp.concatenate(
        [(-2.0 * a).astype(jnp.bfloat16), ones_a, ones_a, a2h, a2l], axis=1
    )
    b_ext = jnp.concatenate(
        [bt.astype(jnp.bfloat16), b2h, b2l, ones_b], axis=0
    )

    d = jax.lax.dot_general(
        a_ext, b_ext,
        dimension_numbers=(((1,), (0,)), ((), ())),
        preferred_element_type=jnp.float32,
    )

    rowacc = jnp.min(d.reshape(BN, M // 128, 128), axis=1)
    rowacc_ref[jax.lax.rem(s, 2)] = rowacc

    part = jnp.min(d.reshape(BN // 32, 4, 8, M), axis=0)
    part32 = part.reshape(32, M)

    @pl.when((s < STEPS) & (n_idx == 0))
    def _():
        colmin_ref[b_par] = part32

    @pl.when((s < STEPS) & (n_idx != 0))
    def _():
        colmin_ref[b_par] = jnp.minimum(colmin_ref[b_par], part32)

    @pl.when(s == 0)
    def _():
        out_ref[...] = jnp.zeros((1, 1), jnp.float32)

    prev = rowacc_ref[jax.lax.rem(s + 1, 2)]
    row_min = jnp.min(prev, axis=1, keepdims=True)
    row_sum = jnp.sum(jnp.sqrt(jnp.maximum(row_min, 0.0)))
    out_ref[...] += jnp.where(s > 0, row_sum, 0.0).reshape(1, 1)

    @pl.when((s > 0) & (n_idx == 0))
    def _():
        pcol = colmin_ref[1 - b_par]
        col_min = jnp.min(pcol, axis=0, keepdims=True)
        col_sum = jnp.sum(jnp.sqrt(jnp.maximum(col_min, 0.0)))
        out_ref[...] += col_sum.reshape(1, 1)


def kernel(array1, array2):
    bt = jnp.transpose(array2, (0, 2, 1))

    def _a_map(s):
        return (jnp.minimum(jax.lax.div(s, NB), B - 1), jax.lax.rem(s, NB), 0)

    def _b_map(s):
        return (jnp.minimum(jax.lax.div(s, NB), B - 1), 0, 0)

    total = pl.pallas_call(
        _chamfer_kernel,
        grid=(STEPS + 1,),
        in_specs=[
            pl.BlockSpec((1, BN, D), _a_map),
            pl.BlockSpec((1, D, M), _b_map),
        ],
        out_specs=pl.BlockSpec((1, 1), lambda s: (0, 0)),
        out_shape=jax.ShapeDtypeStruct((1, 1), jnp.float32),
        scratch_shapes=[
            pltpu.VMEM((2, BN, 128), jnp.float32),
            pltpu.VMEM((2, 32, M), jnp.float32),
        ],
    )(array1, bt)

    return total[0, 0] / (2.0 * B * N)


# speedup vs baseline: 2.2408x; 2.2408x over previous
import jax
import jax.numpy as jnp
from jax.experimental import pallas as pl
from jax.experimental.pallas import tpu as pltpu

B, N, M, D = 4, 4096, 4096, 3
BN = 2048
NB = N // BN


def _chamfer_kernel(a_ref, bt_ref, out_ref, colmin_ref):
    b_idx = pl.program_id(0)
    n_idx = pl.program_id(1)

    a = a_ref[0]
    bt = bt_ref[0]

    a2 = jnp.sum(a * a, axis=1, keepdims=True)
    b2 = jnp.sum(bt * bt, axis=0, keepdims=True)

    a2h = a2.astype(jnp.bfloat16)
    a2l = (a2 - a2h.astype(jnp.float32)).astype(jnp.bfloat16)
    b2h = b2.astype(jnp.bfloat16)
    b2l = (b2 - b2h.astype(jnp.float32)).astype(jnp.bfloat16)

    ones_a = jnp.ones((BN, 1), jnp.bfloat16)
    ones_b = jnp.ones((2, M), jnp.bfloat16)

    a_ext = jnp.concatenate(
        [(-2.0 * a).astype(jnp.bfloat16), ones_a, ones_a, a2h, a2l], axis=1
    )
    b_ext = jnp.concatenate(
        [bt.astype(jnp.bfloat16), b2h, b2l, ones_b], axis=0
    )

    d = jax.lax.dot_general(
        a_ext, b_ext,
        dimension_numbers=(((1,), (0,)), ((), ())),
        preferred_element_type=jnp.float32,
    )

    row_min = jnp.min(d, axis=1, keepdims=True)

    part = jnp.min(d.reshape(BN // 32, 4, 8, M), axis=0)
    col_min = jnp.min(part.reshape(32, M), axis=0, keepdims=True)

    dist1 = jnp.maximum(row_min, 0.0)
    partial = jnp.sum(jnp.sqrt(dist1)).reshape(1, 1)

    @pl.when(n_idx == 0)
    def _():
        colmin_ref[...] = col_min

    @pl.when(n_idx != 0)
    def _():
        colmin_ref[...] = jnp.minimum(colmin_ref[...], col_min)

    @pl.when((b_idx == 0) & (n_idx == 0))
    def _():
        out_ref[...] = jnp.zeros((1, 1), jnp.float32)

    out_ref[...] += partial

    @pl.when(n_idx == NB - 1)
    def _():
        dist2 = jnp.maximum(colmin_ref[...], 0.0)
        out_ref[...] += jnp.sum(jnp.sqrt(dist2)).reshape(1, 1)


def kernel(array1, array2):
    bt = jnp.transpose(array2, (0, 2, 1))

    total = pl.pallas_call(
        _chamfer_kernel,
        grid=(B, NB),
        in_specs=[
            pl.BlockSpec((1, BN, D), lambda b, n: (b, n, 0)),
            pl.BlockSpec((1, D, M), lambda b, n: (b, 0, 0)),
        ],
        out_specs=pl.BlockSpec((1, 1), lambda b, n: (0, 0)),
        out_shape=jax.ShapeDtypeStruct((1, 1), jnp.float32),
        scratch_shapes=[pltpu.VMEM((1, M), jnp.float32)],
    )(array1, bt)

    return total[0, 0] / (2.0 * B * N)
